# Initial kernel scaffold; baseline (speedup 1.0000x reference)
import jax
import jax.numpy as jnp
from jax.experimental import pallas as pl


def kernel(node_type, num_inverted_predecessors, embed_table, W, b):
    raise NotImplementedError("write your pallas kernel here")


# speedup vs baseline: 2.9734x; 2.9734x over previous
import jax
import jax.numpy as jnp
from jax import lax
from jax.experimental import pallas as pl
from jax.experimental.pallas import tpu as pltpu
from jax.experimental.pallas import tpu_sc as plsc

N = 100000
EMBED_DIM = 16
OUT_DIM = 2 * EMBED_DIM
NC, NS = 2, 16
NW = NC * NS
CHUNK = 3136
LAST = N - (NW - 1) * CHUNK
L = 16


def _sc_body(nt_hbm, niv_hbm, tab_hbm, wb_hbm, out_hbm,
             nt_v, niv_v, tab_v, wb_v, out_v):
    c = lax.axis_index("c")
    s = lax.axis_index("s")
    wid = s * NC + c
    is_last = wid == NW - 1
    base = wid * CHUNK

    pltpu.sync_copy(tab_hbm, tab_v)
    pltpu.sync_copy(wb_hbm, wb_v)

    @pl.when(jnp.logical_not(is_last))
    def _():
        pltpu.sync_copy(nt_hbm.at[pl.ds(base, CHUNK)], nt_v)
        pltpu.sync_copy(niv_hbm.at[pl.ds(base, CHUNK)], niv_v)

    @pl.when(is_last)
    def _():
        pltpu.sync_copy(nt_hbm.at[pl.ds(base, LAST)], nt_v.at[pl.ds(0, LAST)])
        pltpu.sync_copy(niv_hbm.at[pl.ds(base, LAST)], niv_v.at[pl.ds(0, LAST)])

    wv = wb_v[pl.ds(0, L)]
    bv = wb_v[pl.ds(L, L)]
    lane = lax.iota(jnp.int32, L)

    nblk = jnp.where(is_last, LAST // L, CHUNK // L)

    def blk_body(i, carry):
        b16 = i * L
        nt16 = nt_v[pl.ds(b16, L)]
        niv16 = niv_v[pl.ds(b16, L)]
        for r in range(L):
            rr = jnp.full((L,), r, dtype=jnp.int32)
            ntb = jnp.take_along_axis(nt16, rr, axis=0)
            nivb = jnp.take_along_axis(niv16, rr, axis=0)
            emb = plsc.load_gather(tab_v, [ntb * EMBED_DIM + lane])
            o = (b16 + r) * OUT_DIM
            out_v[pl.ds(o, L)] = emb
            out_v[pl.ds(o + L, L)] = nivb * wv + bv
        return carry

    lax.fori_loop(0, nblk, blk_body, 0)

    @pl.when(jnp.logical_not(is_last))
    def _():
        pltpu.sync_copy(out_v, out_hbm.at[pl.ds(base * OUT_DIM,
                                                CHUNK * OUT_DIM)])

    @pl.when(is_last)
    def _():
        pltpu.sync_copy(out_v.at[pl.ds(0, LAST * OUT_DIM)],
                        out_hbm.at[pl.ds(base * OUT_DIM, LAST * OUT_DIM)])


@jax.jit
def _sc_call(nt, niv, tab, wb):
    mesh = plsc.VectorSubcoreMesh(
        core_axis_name="c", subcore_axis_name="s",
        num_cores=NC, num_subcores=NS,
    )
    f = pl.kernel(
        _sc_body,
        out_type=jax.ShapeDtypeStruct((N * OUT_DIM,), jnp.float32),
        mesh=mesh,
        compiler_params=pltpu.CompilerParams(needs_layout_passes=False),
        scratch_types=[
            pltpu.VMEM((CHUNK,), jnp.int32),
            pltpu.VMEM((CHUNK,), jnp.float32),
            pltpu.VMEM((3 * EMBED_DIM,), jnp.float32),
            pltpu.VMEM((2 * EMBED_DIM,), jnp.float32),
            pltpu.VMEM((CHUNK * OUT_DIM,), jnp.float32),
        ],
    )
    return f(nt, niv, tab, wb)


def kernel(node_type, num_inverted_predecessors, embed_table, W, b):
    nt = node_type.astype(jnp.int32)
    wb = jnp.concatenate([W.T, b[None, :]], axis=0)
    out_flat = _sc_call(nt, num_inverted_predecessors,
                        embed_table.reshape(-1), wb.reshape(-1))
    return out_flat.reshape(N, OUT_DIM)


# speedup vs baseline: 3.7867x; 1.2735x over previous
import jax
import jax.numpy as jnp
from jax import lax
from jax.experimental import pallas as pl
from jax.experimental.pallas import tpu as pltpu
from jax.experimental.pallas import tpu_sc as plsc

N = 100000
EMBED_DIM = 16
OUT_DIM = 2 * EMBED_DIM
NC, NS = 2, 16
NW = NC * NS
ROWS_W = 3200
ROWS_LAST = N - (NW - 1) * ROWS_W
SUB = 400
TSUB = SUB // 8
NSUB = ROWS_W // SUB
NSUB_LAST = ROWS_LAST // SUB
L = 16


def _sc_body(nt_hbm, niv_hbm, tab_hbm, wb_hbm, out_hbm,
             nt_v, niv_v, tab_v, wb_v, out_v):
    c = lax.axis_index("c")
    s = lax.axis_index("s")
    wid = s * NC + c
    is_last = wid == NW - 1
    base = wid * ROWS_W

    pltpu.sync_copy(tab_hbm, tab_v)
    pltpu.sync_copy(wb_hbm, wb_v)

    @pl.when(jnp.logical_not(is_last))
    def _():
        pltpu.sync_copy(nt_hbm.at[pl.ds(base, ROWS_W)], nt_v)
        pltpu.sync_copy(niv_hbm.at[pl.ds(base, ROWS_W)], niv_v)

    @pl.when(is_last)
    def _():
        pltpu.sync_copy(nt_hbm.at[pl.ds(base, ROWS_LAST)],
                        nt_v.at[pl.ds(0, ROWS_LAST)])
        pltpu.sync_copy(niv_hbm.at[pl.ds(base, ROWS_LAST)],
                        niv_v.at[pl.ds(0, ROWS_LAST)])

    wv = wb_v[pl.ds(0, L)]
    bv = wb_v[pl.ds(L, L)]
    lane = lax.iota(jnp.int32, L)

    nsub = jnp.where(is_last, NSUB_LAST, NSUB)

    def sub_body(k, carry):
        def blk_body(i, carry2):
            b16 = k * SUB + i * L
            nt16 = nt_v[pl.ds(b16, L)]
            niv16 = niv_v[pl.ds(b16, L)]
            for r in range(L):
                rr = jnp.full((L,), r, dtype=jnp.int32)
                ntb = jnp.take_along_axis(nt16, rr, axis=0)
                nivb = jnp.take_along_axis(niv16, rr, axis=0)
                emb = plsc.load_gather(tab_v, [ntb * EMBED_DIM + lane])
                t = 2 * i + (r // 8)
                sl = r % 8
                out_v[t, sl, pl.ds(0, L)] = emb
                out_v[t, sl, pl.ds(L, L)] = nivb * wv + bv
            return carry2

        lax.fori_loop(0, SUB // L, blk_body, 0)
        tile_base = (base // 8) + k * TSUB
        pltpu.sync_copy(out_v, out_hbm.at[pl.ds(tile_base, TSUB), :, :])
        return carry

    lax.fori_loop(0, nsub, sub_body, 0)


@jax.jit
def _sc_call(nt, niv, tab, wb):
    mesh = plsc.VectorSubcoreMesh(
        core_axis_name="c", subcore_axis_name="s",
        num_cores=NC, num_subcores=NS,
    )
    f = pl.kernel(
        _sc_body,
        out_type=jax.ShapeDtypeStruct((N // 8, 8, OUT_DIM), jnp.float32),
        mesh=mesh,
        compiler_params=pltpu.CompilerParams(needs_layout_passes=False),
        scratch_types=[
            pltpu.VMEM((ROWS_W,), jnp.int32),
            pltpu.VMEM((ROWS_W,), jnp.float32),
            pltpu.VMEM((3 * EMBED_DIM,), jnp.float32),
            pltpu.VMEM((2 * EMBED_DIM,), jnp.float32),
            pltpu.VMEM((TSUB, 8, OUT_DIM), jnp.float32),
        ],
    )
    return f(nt, niv, tab, wb)


def kernel(node_type, num_inverted_predecessors, embed_table, W, b):
    nt = node_type.astype(jnp.int32)
    wb = jnp.concatenate([W.T, b[None, :]], axis=0)
    out3 = _sc_call(nt, num_inverted_predecessors,
                    embed_table.reshape(-1), wb.reshape(-1))
    return out3.reshape(N, OUT_DIM)
